# baseline (speedup 1.0000x reference)
import jax
import jax.numpy as jnp
from jax.experimental import pallas as pl
from jax.experimental.pallas import tpu as pltpu

_TBLK = 512


def _body(h_ref, w_ref, out_ref, carry_ref):
    t = pl.program_id(1)
    h = h_ref[0]
    g = jnp.dot(h, w_ref[...], preferred_element_type=jnp.float32)
    av = g[:, 0:1]
    cv = g[:, 1:2]
    prev = jnp.where(t == 0, 0.0, carry_ref[0])
    shifted = jnp.roll(cv, 1, axis=0)
    row = jax.lax.broadcasted_iota(jnp.int32, cv.shape, 0)
    out_ref[0] = av + jnp.where(row == 0, prev, shifted)
    carry_ref[0] = cv[cv.shape[0] - 1, 0]


@jax.jit
def kernel(hidden_states, W):
    b, t, d = hidden_states.shape
    nt = t // _TBLK
    wk = W.reshape(2, d).T
    out = pl.pallas_call(
        _body,
        grid=(b, nt),
        in_specs=[
            pl.BlockSpec((1, _TBLK, d), lambda i, j: (i, j, 0)),
            pl.BlockSpec((d, 2), lambda i, j: (0, 0)),
        ],
        out_specs=pl.BlockSpec((1, _TBLK, 1), lambda i, j: (i * nt + j, 0, 0)),
        out_shape=jax.ShapeDtypeStruct((b * nt, _TBLK, 1), hidden_states.dtype),
        scratch_shapes=[pltpu.SMEM((1,), jnp.float32)],
    )(hidden_states, wk)
    return out.reshape(b, t)


# speedup vs baseline: 1.2976x; 1.2976x over previous
import jax
import jax.numpy as jnp
from jax.experimental import pallas as pl
from jax.experimental.pallas import tpu as pltpu

_TBLK = 1024


def _body(h_ref, w_ref, out_ref, carry_ref):
    t = pl.program_id(1)
    h = h_ref[0]
    d = h.shape[-1]
    w1 = w_ref[:, :d]
    w2 = w_ref[:, d:]
    av = jnp.sum(h * w1, axis=1)[None, :]
    cv = jnp.sum(h * w2, axis=1)[None, :]
    prev = jnp.where(t == 0, 0.0, carry_ref[0])
    shifted = jnp.roll(cv, 1, axis=1)
    col = jax.lax.broadcasted_iota(jnp.int32, cv.shape, 1)
    out_ref[0] = av + jnp.where(col == 0, prev, shifted)
    carry_ref[0] = cv[0, cv.shape[1] - 1]


@jax.jit
def kernel(hidden_states, W):
    b, t, d = hidden_states.shape
    nt = t // _TBLK
    out = pl.pallas_call(
        _body,
        grid=(b, nt),
        in_specs=[
            pl.BlockSpec((1, _TBLK, d), lambda i, j: (i, j, 0)),
            pl.BlockSpec((1, 2 * d), lambda i, j: (0, 0)),
        ],
        out_specs=pl.BlockSpec((1, 1, _TBLK), lambda i, j: (i * nt + j, 0, 0)),
        out_shape=jax.ShapeDtypeStruct((b * nt, 1, _TBLK), hidden_states.dtype),
        scratch_shapes=[pltpu.SMEM((1,), jnp.float32)],
        compiler_params=pltpu.CompilerParams(
            dimension_semantics=("parallel", "arbitrary"),
        ),
    )(hidden_states, W)
    return out.reshape(b, t)


# speedup vs baseline: 1.4154x; 1.0908x over previous
import jax
import jax.numpy as jnp
from jax.experimental import pallas as pl
from jax.experimental.pallas import tpu as pltpu

_TBLK = 2048


def _body(h_ref, w_ref, out_ref, carry_ref):
    t = pl.program_id(1)
    h = h_ref[0]
    d = h.shape[-1]
    w1 = w_ref[:, :d]
    w2 = w_ref[:, d:]
    av = jnp.sum(h * w1, axis=1)[None, :]
    cv = jnp.sum(h * w2, axis=1)[None, :]
    prev = jnp.where(t == 0, 0.0, carry_ref[0])
    shifted = jnp.roll(cv, 1, axis=1)
    col = jax.lax.broadcasted_iota(jnp.int32, cv.shape, 1)
    out_ref[0] = av + jnp.where(col == 0, prev, shifted)
    carry_ref[0] = cv[0, cv.shape[1] - 1]


@jax.jit
def kernel(hidden_states, W):
    b, t, d = hidden_states.shape
    nt = t // _TBLK
    out = pl.pallas_call(
        _body,
        grid=(b, nt),
        in_specs=[
            pl.BlockSpec((1, _TBLK, d), lambda i, j: (i, j, 0)),
            pl.BlockSpec((1, 2 * d), lambda i, j: (0, 0)),
        ],
        out_specs=pl.BlockSpec((1, 1, _TBLK), lambda i, j: (i * nt + j, 0, 0)),
        out_shape=jax.ShapeDtypeStruct((b * nt, 1, _TBLK), hidden_states.dtype),
        scratch_shapes=[pltpu.SMEM((1,), jnp.float32)],
        compiler_params=pltpu.CompilerParams(
            dimension_semantics=("parallel", "arbitrary"),
        ),
    )(hidden_states, W)
    return out.reshape(b, t)
